# baseline (speedup 1.0000x reference)
import functools

import jax
import jax.numpy as jnp
from jax import lax
from jax.experimental import pallas as pl
from jax.experimental.pallas import tpu as pltpu
from jax.experimental.pallas import tpu_sc as plsc

N = 10000
E = 320000
L = 128
H = 64

RB = 1000
NSTEPS = N // RB

NC = 2
NS = 16
EPT = E // NS
C = 80
NCH = EPT // C
NP = 10240
RPT = NP // NS


def _enc_body(nfb_ref, lat_ref, wn0_ref, wn1_ref, we_ref, wm1_ref, wm2_ref,
              wm3_ref, ne_ref, a_ref, b_ref, vpn_ref):
    ne = jnp.maximum(
        nfb_ref[...] * wn0_ref[...]
        + jnp.dot(lat_ref[...], wn1_ref[...], preferred_element_type=jnp.float32),
        0.0)
    ne_ref[...] = ne
    a_ref[...] = jnp.dot(ne, wm1_ref[...], preferred_element_type=jnp.float32)
    b_ref[...] = jnp.dot(ne, wm2_ref[...], preferred_element_type=jnp.float32)
    ep = jnp.maximum(we_ref[...], 0.0)
    en = jnp.maximum(-we_ref[...], 0.0)
    vp = jnp.dot(ep, wm3_ref[...], preferred_element_type=jnp.float32)
    vn = jnp.dot(en, wm3_ref[...], preferred_element_type=jnp.float32)
    vpn_ref[...] = jnp.concatenate([vp, vn], axis=0)


def _encode(nf_b, lat, wn0, wn1, we, wm1, wm2, wm3):
    row = pl.BlockSpec((RB, L), lambda i: (i, 0))
    w1 = pl.BlockSpec((1, L), lambda i: (0, 0))
    wL = pl.BlockSpec((L, L), lambda i: (0, 0))
    return pl.pallas_call(
        _enc_body,
        grid=(NSTEPS,),
        in_specs=[row, row, w1, wL, w1, wL, wL, wL],
        out_specs=[row, row, row, pl.BlockSpec((2, L), lambda i: (0, 0))],
        out_shape=[
            jax.ShapeDtypeStruct((N, L), jnp.float32),
            jax.ShapeDtypeStruct((N, L), jnp.float32),
            jax.ShapeDtypeStruct((N, L), jnp.float32),
            jax.ShapeDtypeStruct((2, L), jnp.float32),
        ],
    )(nf_b, lat, wn0, wn1, we, wm1, wm2, wm3)


def _edge_body(a_hbm, b_hbm, src_hbm, dst_hbm, ef_hbm, vpn_hbm, zer_hbm,
               out0_hbm, out1_hbm,
               agg_sp,
               sidx0, sidx1, didx0, didx1, sg0, sg1, dg0, dg1,
               dsc0, dsc1, ef0, ef1,
               ar0, ar1, br0, br1, mg0, mg1, vpn_v,
               sem_i0, sem_i1, sem_a0, sem_a1, sem_b0, sem_b1,
               sem_s0, sem_s1, sem_z):
    sidx = (sidx0, sidx1)
    didx = (didx0, didx1)
    sg = (sg0, sg1)
    dg = (dg0, dg1)
    dsc = (dsc0, dsc1)
    efv = (ef0, ef1)
    ar = (ar0, ar1)
    br = (br0, br1)
    mg = (mg0, mg1)
    sem_i = (sem_i0, sem_i1)
    sem_a = (sem_a0, sem_a1)
    sem_b = (sem_b0, sem_b1)
    sem_s = (sem_s0, sem_s1)

    cid = lax.axis_index("c")
    sid = lax.axis_index("s")
    ebase = sid * EPT
    goff = cid * N

    pltpu.async_copy(zer_hbm, agg_sp.at[pl.ds(sid * RPT, RPT)], sem_z).wait()
    pltpu.sync_copy(vpn_hbm.at[pl.ds(cid * 2, 2)], vpn_v)
    plsc.subcore_barrier()

    def issue_idx(i, b):
        base = ebase + i * C
        pltpu.async_copy(src_hbm.at[pl.ds(base, C)], sidx[b], sem_i[b])
        pltpu.async_copy(dst_hbm.at[pl.ds(base, C)], didx[b], sem_i[b])
        pltpu.async_copy(ef_hbm.at[pl.ds(base, C)], efv[b], sem_i[b])

    def wait_idx(b):
        pltpu.make_async_copy(src_hbm.at[pl.ds(0, C)], sidx[b], sem_i[b]).wait()
        pltpu.make_async_copy(dst_hbm.at[pl.ds(0, C)], didx[b], sem_i[b]).wait()
        pltpu.make_async_copy(ef_hbm.at[pl.ds(0, C)], efv[b], sem_i[b]).wait()

    def adjust_idx(b):
        for q in range(C // 16):
            sl = pl.ds(q * 16, 16)
            sg[b][sl] = sidx[b][sl] + goff
            dg[b][sl] = didx[b][sl] + goff

    def issue_gathers(b):
        pltpu.async_copy(a_hbm.at[sg[b]], ar[b], sem_a[b])
        pltpu.async_copy(b_hbm.at[dg[b]], br[b], sem_b[b])

    def wait_gathers(b):
        pltpu.make_async_copy(a_hbm.at[sg[b]], ar[b], sem_a[b]).wait()
        pltpu.make_async_copy(b_hbm.at[dg[b]], br[b], sem_b[b]).wait()

    def wait_scatter(b):
        pltpu.make_async_copy(mg[b], agg_sp.at[dsc[b]], sem_s[b]).wait()

    def compute(b):
        arb, brb, efb, mgb = ar[b], br[b], efv[b], mg[b]

        def edge16(q):
            ev = efb[pl.ds(q * 16, 16)]
            spv = jnp.maximum(ev, 0.0)
            snv = jnp.maximum(-ev, 0.0)
            for rr in range(16):
                lane = jnp.full((16,), rr, jnp.int32)
                sp = spv.at[lane].get(mode="promise_in_bounds")
                sn = snv.at[lane].get(mode="promise_in_bounds")
                r = q * 16 + rr
                for j in range(H // 16):
                    sl = pl.ds(j * 16, 16)
                    v = (arb[r, sl] + brb[r, sl]
                         + sp * vpn_v[0, sl] + sn * vpn_v[1, sl])
                    mgb[r, sl] = jnp.maximum(v, 0.0)

        plsc.parallel_loop(0, C // 16, 1, unroll=1)(edge16)

    def body(i, b):
        o = 1 - b

        @pl.when(i >= 1)
        def _():
            wait_scatter(o)

        @pl.when(i + 1 < NCH)
        def _():
            wait_idx(o)
            adjust_idx(o)
            issue_gathers(o)

        wait_gathers(b)
        compute(b)
        for q in range(C // 16):
            sl = pl.ds(q * 16, 16)
            dsc[b][sl] = didx[b][sl]
        pltpu.async_copy(mg[b], agg_sp.at[dsc[b]], sem_s[b], add=True)

        @pl.when(i + 2 < NCH)
        def _():
            issue_idx(i + 2, b)

    issue_idx(0, 0)
    issue_idx(1, 1)
    wait_idx(0)
    adjust_idx(0)
    issue_gathers(0)

    def pair(t, carry):
        body(2 * t, 0)
        body(2 * t + 1, 1)
        return carry

    lax.fori_loop(0, NCH // 2, pair, 0)
    wait_scatter(1)
    plsc.subcore_barrier()

    rows = agg_sp.at[pl.ds(sid * RPT, RPT)]

    @pl.when(cid == 0)
    def _():
        pltpu.sync_copy(rows, out0_hbm.at[pl.ds(sid * RPT, RPT)])

    @pl.when(cid == 1)
    def _():
        pltpu.sync_copy(rows, out1_hbm.at[pl.ds(sid * RPT, RPT)])


_edge_call = functools.partial(
    pl.kernel,
    out_type=(
        jax.ShapeDtypeStruct((NP, H), jnp.float32),
        jax.ShapeDtypeStruct((NP, H), jnp.float32),
    ),
    mesh=plsc.VectorSubcoreMesh(
        core_axis_name="c", subcore_axis_name="s",
        num_cores=NC, num_subcores=NS),
    compiler_params=pltpu.CompilerParams(use_tc_tiling_on_sc=False),
    scratch_types=(
        [pltpu.VMEM_SHARED((NP, H), jnp.float32)]
        + [pltpu.VMEM((C,), jnp.int32)] * 10
        + [pltpu.VMEM((C,), jnp.float32)] * 2
        + [pltpu.VMEM((C, H), jnp.float32)] * 6
        + [pltpu.VMEM((2, H), jnp.float32)]
        + [pltpu.SemaphoreType.DMA] * 9
    ),
)(_edge_body)


def _dec_body(ne_ref, g0_ref, g1_ref, wu1_ref, wu2a_ref, wu2b_ref,
              wd1a_ref, wd1b_ref, wd2_ref, out_ref):
    ne = ne_ref[...]
    lo = jnp.maximum(
        jnp.dot(ne, wu1_ref[...], preferred_element_type=jnp.float32)
        + jnp.dot(g0_ref[...], wu2a_ref[...], preferred_element_type=jnp.float32)
        + jnp.dot(g1_ref[...], wu2b_ref[...], preferred_element_type=jnp.float32),
        0.0)
    h = jnp.maximum(
        jnp.dot(ne, wd1a_ref[...], preferred_element_type=jnp.float32)
        + jnp.dot(lo, wd1b_ref[...], preferred_element_type=jnp.float32), 0.0)
    out_ref[...] = jnp.dot(h, wd2_ref[...], preferred_element_type=jnp.float32)


def _decode(ne, g0, g1, wu1, wu2a, wu2b, wd1a, wd1b, wd2p):
    row = pl.BlockSpec((RB, L), lambda i: (i, 0))
    half = pl.BlockSpec((RB, H), lambda i: (i, 0))
    wL = pl.BlockSpec((L, L), lambda i: (0, 0))
    wH = pl.BlockSpec((H, L), lambda i: (0, 0))
    return pl.pallas_call(
        _dec_body,
        grid=(NSTEPS,),
        in_specs=[row, half, half, wL, wH, wH, wL, wL, wL],
        out_specs=row,
        out_shape=jax.ShapeDtypeStruct((N, L), jnp.float32),
    )(ne, g0, g1, wu1, wu2a, wu2b, wd1a, wd1b, wd2p)


def kernel(node_features, edge_features, latent_features, edge_index,
           W_node, W_edge, W_msg, W_upd, W_dec1, W_dec2):
    nf_b = jnp.broadcast_to(
        node_features.astype(jnp.float32)[:, None], (N, L))
    lat = latent_features.astype(jnp.float32)
    ne, a, b, vpn = _encode(
        nf_b, lat, W_node[0:1], W_node[1:], W_edge,
        W_msg[0:L], W_msg[L:2 * L], W_msg[2 * L:])
    a2 = jnp.concatenate([a[:, :H], a[:, H:]], axis=0)
    b2 = jnp.concatenate([b[:, :H], b[:, H:]], axis=0)
    vpnr = jnp.stack([vpn[0, :H], vpn[1, :H], vpn[0, H:], vpn[1, H:]])
    src = edge_index[0].astype(jnp.int32)
    dst = edge_index[1].astype(jnp.int32)
    ef = edge_features.astype(jnp.float32)
    zer = jnp.zeros((RPT, H), jnp.float32)
    g0, g1 = _edge_call(a2, b2, src, dst, ef, vpnr, zer)
    wd2p = jnp.pad(W_dec2, ((0, 0), (0, L - 1)))
    outp = _decode(ne, g0, g1, W_upd[:L], W_upd[L:L + H], W_upd[L + H:],
                   W_dec1[:L], W_dec1[L:], wd2p)
    return outp[:, :1]


# speedup vs baseline: 2.0558x; 2.0558x over previous
import functools

import jax
import jax.numpy as jnp
from jax import lax
from jax.experimental import pallas as pl
from jax.experimental.pallas import tpu as pltpu
from jax.experimental.pallas import tpu_sc as plsc

N = 10000
E = 320000
L = 128
H = 64

RB = 1000
NSTEPS = N // RB

NC = 2
NS = 16
EPT = E // NS
C = 80
NCH = EPT // C
NP = 10240
RPT = NP // NS


def _enc_body(nfb_ref, lat_ref, wn0_ref, wn1_ref, we_ref, wm1_ref, wm2_ref,
              wm3_ref, ne_ref, a_ref, b_ref, vpn_ref):
    ne = jnp.maximum(
        nfb_ref[...] * wn0_ref[...]
        + jnp.dot(lat_ref[...], wn1_ref[...], preferred_element_type=jnp.float32),
        0.0)
    ne_ref[...] = ne
    a_ref[...] = jnp.dot(ne, wm1_ref[...], preferred_element_type=jnp.float32)
    b_ref[...] = jnp.dot(ne, wm2_ref[...], preferred_element_type=jnp.float32)
    ep = jnp.maximum(we_ref[...], 0.0)
    en = jnp.maximum(-we_ref[...], 0.0)
    vp = jnp.dot(ep, wm3_ref[...], preferred_element_type=jnp.float32)
    vn = jnp.dot(en, wm3_ref[...], preferred_element_type=jnp.float32)
    vpn_ref[...] = jnp.concatenate([vp, vn], axis=0)


def _encode(nf_b, lat, wn0, wn1, we, wm1, wm2, wm3):
    row = pl.BlockSpec((RB, L), lambda i: (i, 0))
    w1 = pl.BlockSpec((1, L), lambda i: (0, 0))
    wL = pl.BlockSpec((L, L), lambda i: (0, 0))
    return pl.pallas_call(
        _enc_body,
        grid=(NSTEPS,),
        in_specs=[row, row, w1, wL, w1, wL, wL, wL],
        out_specs=[row, row, row, pl.BlockSpec((2, L), lambda i: (0, 0))],
        out_shape=[
            jax.ShapeDtypeStruct((N, L), jnp.float32),
            jax.ShapeDtypeStruct((N, L), jnp.float32),
            jax.ShapeDtypeStruct((N, L), jnp.float32),
            jax.ShapeDtypeStruct((2, L), jnp.float32),
        ],
    )(nf_b, lat, wn0, wn1, we, wm1, wm2, wm3)


def _edge_body(a_hbm, b_hbm, src_hbm, dst_hbm, ef_hbm, vpn_hbm, zer_hbm,
               out0_hbm, out1_hbm,
               agg_sp,
               sidx0, sidx1, didx0, didx1, sg0, sg1, dg0, dg1,
               dsc0, dsc1, ef0, ef1,
               ar0, ar1, br0, br1, mg0, mg1, vpn_v,
               sem_i0, sem_i1, sem_a0, sem_a1, sem_b0, sem_b1,
               sem_s0, sem_s1, sem_z):
    sidx = (sidx0, sidx1)
    didx = (didx0, didx1)
    sg = (sg0, sg1)
    dg = (dg0, dg1)
    dsc = (dsc0, dsc1)
    efv = (ef0, ef1)
    ar = (ar0, ar1)
    br = (br0, br1)
    mg = (mg0, mg1)
    sem_i = (sem_i0, sem_i1)
    sem_a = (sem_a0, sem_a1)
    sem_b = (sem_b0, sem_b1)
    sem_s = (sem_s0, sem_s1)

    cid = lax.axis_index("c")
    sid = lax.axis_index("s")
    ebase = sid * EPT
    goff = cid * N

    pltpu.async_copy(zer_hbm, agg_sp.at[pl.ds(sid * RPT, RPT)], sem_z).wait()
    pltpu.sync_copy(vpn_hbm.at[pl.ds(cid * 2, 2)], vpn_v)
    plsc.subcore_barrier()
    vps = tuple(vpn_v[0, pl.ds(j * 16, 16)] for j in range(H // 16))
    vns = tuple(vpn_v[1, pl.ds(j * 16, 16)] for j in range(H // 16))

    def issue_idx(i, b):
        base = ebase + i * C
        pltpu.async_copy(src_hbm.at[pl.ds(base, C)], sidx[b], sem_i[b])
        pltpu.async_copy(dst_hbm.at[pl.ds(base, C)], didx[b], sem_i[b])
        pltpu.async_copy(ef_hbm.at[pl.ds(base, C)], efv[b], sem_i[b])

    def wait_idx(b):
        pltpu.make_async_copy(src_hbm.at[pl.ds(0, C)], sidx[b], sem_i[b]).wait()
        pltpu.make_async_copy(dst_hbm.at[pl.ds(0, C)], didx[b], sem_i[b]).wait()
        pltpu.make_async_copy(ef_hbm.at[pl.ds(0, C)], efv[b], sem_i[b]).wait()

    def adjust_idx(b):
        for q in range(C // 16):
            sl = pl.ds(q * 16, 16)
            sg[b][sl] = sidx[b][sl] + goff
            dg[b][sl] = didx[b][sl] + goff

    def issue_gathers(b):
        pltpu.async_copy(a_hbm.at[sg[b]], ar[b], sem_a[b])
        pltpu.async_copy(b_hbm.at[dg[b]], br[b], sem_b[b])

    def wait_gathers(b):
        pltpu.make_async_copy(a_hbm.at[sg[b]], ar[b], sem_a[b]).wait()
        pltpu.make_async_copy(b_hbm.at[dg[b]], br[b], sem_b[b]).wait()

    def wait_scatter(b):
        pltpu.make_async_copy(mg[b], agg_sp.at[dsc[b]], sem_s[b]).wait()

    def compute(b):
        arb, brb, efb, mgb = ar[b], br[b], efv[b], mg[b]

        def grp(q):
            ev = efb[pl.ds(q * 16, 16)]
            spv = jnp.maximum(ev, 0.0)
            snv = jnp.maximum(-ev, 0.0)

            def edge(rr):
                lane = jnp.full((16,), 0, jnp.int32) + rr
                sp = spv.at[lane].get(mode="promise_in_bounds")
                sn = snv.at[lane].get(mode="promise_in_bounds")
                e = q * 16 + rr
                for j in range(H // 16):
                    sl = pl.ds(j * 16, 16)
                    v = arb[e, sl] + brb[e, sl] + sp * vps[j] + sn * vns[j]
                    mgb[e, sl] = jnp.maximum(v, 0.0)

            plsc.parallel_loop(0, 16, 1, unroll=2)(edge)

        plsc.parallel_loop(0, C // 16, 1, unroll=1)(grp)

    def body(i, b):
        o = 1 - b

        @pl.when(i >= 1)
        def _():
            wait_scatter(o)

        @pl.when(i + 1 < NCH)
        def _():
            wait_idx(o)
            adjust_idx(o)
            issue_gathers(o)

        wait_gathers(b)
        compute(b)
        for q in range(C // 16):
            sl = pl.ds(q * 16, 16)
            dsc[b][sl] = didx[b][sl]
        pltpu.async_copy(mg[b], agg_sp.at[dsc[b]], sem_s[b], add=True)

        @pl.when(i + 2 < NCH)
        def _():
            issue_idx(i + 2, b)

    issue_idx(0, 0)
    issue_idx(1, 1)
    wait_idx(0)
    adjust_idx(0)
    issue_gathers(0)

    def pair(t, carry):
        body(2 * t, 0)
        body(2 * t + 1, 1)
        return carry

    lax.fori_loop(0, NCH // 2, pair, 0)
    wait_scatter(1)
    plsc.subcore_barrier()

    rows = agg_sp.at[pl.ds(sid * RPT, RPT)]

    @pl.when(cid == 0)
    def _():
        pltpu.sync_copy(rows, out0_hbm.at[pl.ds(sid * RPT, RPT)])

    @pl.when(cid == 1)
    def _():
        pltpu.sync_copy(rows, out1_hbm.at[pl.ds(sid * RPT, RPT)])


_edge_call = functools.partial(
    pl.kernel,
    out_type=(
        jax.ShapeDtypeStruct((NP, H), jnp.float32),
        jax.ShapeDtypeStruct((NP, H), jnp.float32),
    ),
    mesh=plsc.VectorSubcoreMesh(
        core_axis_name="c", subcore_axis_name="s",
        num_cores=NC, num_subcores=NS),
    compiler_params=pltpu.CompilerParams(use_tc_tiling_on_sc=False),
    scratch_types=(
        [pltpu.VMEM_SHARED((NP, H), jnp.float32)]
        + [pltpu.VMEM((C,), jnp.int32)] * 10
        + [pltpu.VMEM((C,), jnp.float32)] * 2
        + [pltpu.VMEM((C, H), jnp.float32)] * 6
        + [pltpu.VMEM((2, H), jnp.float32)]
        + [pltpu.SemaphoreType.DMA] * 9
    ),
)(_edge_body)


def _dec_body(ne_ref, g0_ref, g1_ref, wu1_ref, wu2a_ref, wu2b_ref,
              wd1a_ref, wd1b_ref, wd2_ref, out_ref):
    ne = ne_ref[...]
    lo = jnp.maximum(
        jnp.dot(ne, wu1_ref[...], preferred_element_type=jnp.float32)
        + jnp.dot(g0_ref[...], wu2a_ref[...], preferred_element_type=jnp.float32)
        + jnp.dot(g1_ref[...], wu2b_ref[...], preferred_element_type=jnp.float32),
        0.0)
    h = jnp.maximum(
        jnp.dot(ne, wd1a_ref[...], preferred_element_type=jnp.float32)
        + jnp.dot(lo, wd1b_ref[...], preferred_element_type=jnp.float32), 0.0)
    out_ref[...] = jnp.dot(h, wd2_ref[...], preferred_element_type=jnp.float32)


def _decode(ne, g0, g1, wu1, wu2a, wu2b, wd1a, wd1b, wd2p):
    row = pl.BlockSpec((RB, L), lambda i: (i, 0))
    half = pl.BlockSpec((RB, H), lambda i: (i, 0))
    wL = pl.BlockSpec((L, L), lambda i: (0, 0))
    wH = pl.BlockSpec((H, L), lambda i: (0, 0))
    return pl.pallas_call(
        _dec_body,
        grid=(NSTEPS,),
        in_specs=[row, half, half, wL, wH, wH, wL, wL, wL],
        out_specs=row,
        out_shape=jax.ShapeDtypeStruct((N, L), jnp.float32),
    )(ne, g0, g1, wu1, wu2a, wu2b, wd1a, wd1b, wd2p)


def kernel(node_features, edge_features, latent_features, edge_index,
           W_node, W_edge, W_msg, W_upd, W_dec1, W_dec2):
    nf_b = jnp.broadcast_to(
        node_features.astype(jnp.float32)[:, None], (N, L))
    lat = latent_features.astype(jnp.float32)
    ne, a, b, vpn = _encode(
        nf_b, lat, W_node[0:1], W_node[1:], W_edge,
        W_msg[0:L], W_msg[L:2 * L], W_msg[2 * L:])
    a2 = jnp.concatenate([a[:, :H], a[:, H:]], axis=0)
    b2 = jnp.concatenate([b[:, :H], b[:, H:]], axis=0)
    vpnr = jnp.stack([vpn[0, :H], vpn[1, :H], vpn[0, H:], vpn[1, H:]])
    src = edge_index[0].astype(jnp.int32)
    dst = edge_index[1].astype(jnp.int32)
    ef = edge_features.astype(jnp.float32)
    zer = jnp.zeros((RPT, H), jnp.float32)
    g0, g1 = _edge_call(a2, b2, src, dst, ef, vpnr, zer)
    wd2p = jnp.pad(W_dec2, ((0, 0), (0, L - 1)))
    outp = _decode(ne, g0, g1, W_upd[:L], W_upd[L:L + H], W_upd[L + H:],
                   W_dec1[:L], W_dec1[L:], wd2p)
    return outp[:, :1]


# speedup vs baseline: 2.2729x; 1.1056x over previous
import functools

import jax
import jax.numpy as jnp
from jax import lax
from jax.experimental import pallas as pl
from jax.experimental.pallas import tpu as pltpu
from jax.experimental.pallas import tpu_sc as plsc

N = 10000
E = 320000
L = 128
H = 64

RB = 1000
NSTEPS = N // RB

NC = 2
NS = 16
EPT = E // NS
C = 160
CH = 80
NCH = EPT // C
NP = 10240
RPT = NP // NS


def _enc_body(nfb_ref, lat_ref, wn0_ref, wn1_ref, we_ref, wm1_ref, wm2_ref,
              wm3_ref, ne_ref, a_ref, b_ref, vpn_ref):
    ne = jnp.maximum(
        nfb_ref[...] * wn0_ref[...]
        + jnp.dot(lat_ref[...], wn1_ref[...], preferred_element_type=jnp.float32),
        0.0)
    ne_ref[...] = ne
    a_ref[...] = jnp.dot(ne, wm1_ref[...], preferred_element_type=jnp.float32)
    b_ref[...] = jnp.dot(ne, wm2_ref[...], preferred_element_type=jnp.float32)
    ep = jnp.maximum(we_ref[...], 0.0)
    en = jnp.maximum(-we_ref[...], 0.0)
    vp = jnp.dot(ep, wm3_ref[...], preferred_element_type=jnp.float32)
    vn = jnp.dot(en, wm3_ref[...], preferred_element_type=jnp.float32)
    vpn_ref[...] = jnp.concatenate([vp, vn], axis=0)


def _encode(nf_b, lat, wn0, wn1, we, wm1, wm2, wm3):
    row = pl.BlockSpec((RB, L), lambda i: (i, 0))
    w1 = pl.BlockSpec((1, L), lambda i: (0, 0))
    wL = pl.BlockSpec((L, L), lambda i: (0, 0))
    return pl.pallas_call(
        _enc_body,
        grid=(NSTEPS,),
        in_specs=[row, row, w1, wL, w1, wL, wL, wL],
        out_specs=[row, row, row, pl.BlockSpec((2, L), lambda i: (0, 0))],
        out_shape=[
            jax.ShapeDtypeStruct((N, L), jnp.float32),
            jax.ShapeDtypeStruct((N, L), jnp.float32),
            jax.ShapeDtypeStruct((N, L), jnp.float32),
            jax.ShapeDtypeStruct((2, L), jnp.float32),
        ],
    )(nf_b, lat, wn0, wn1, we, wm1, wm2, wm3)


def _edge_body(a_hbm, b_hbm, ed_hbm, ef_hbm, vpn_hbm, zer_hbm,
               out0_hbm, out1_hbm,
               agg_sp,
               ed0, ed1, ef0, ef1,
               sga0, sga1, sgb0, sgb1, dga0, dga1, dgb0, dgb1,
               dsa0, dsa1, dsb0, dsb1,
               ar0, ar1, br0, br1, mg0, mg1, vpn_v,
               sem_i0, sem_i1, sem_a0, sem_a1, sem_b0, sem_b1,
               sem_s0, sem_s1, sem_z):
    ed = (ed0, ed1)
    efv = (ef0, ef1)
    sga = (sga0, sga1)
    sgb = (sgb0, sgb1)
    dga = (dga0, dga1)
    dgb = (dgb0, dgb1)
    dsa = (dsa0, dsa1)
    dsb = (dsb0, dsb1)
    ar = (ar0, ar1)
    br = (br0, br1)
    mg = (mg0, mg1)
    sem_i = (sem_i0, sem_i1)
    sem_a = (sem_a0, sem_a1)
    sem_b = (sem_b0, sem_b1)
    sem_s = (sem_s0, sem_s1)

    cid = lax.axis_index("c")
    sid = lax.axis_index("s")
    ebase = sid * NCH * 2 * C
    efbase = sid * EPT
    goff = cid * N

    pltpu.async_copy(zer_hbm, agg_sp.at[pl.ds(sid * RPT, RPT)], sem_z).wait()
    pltpu.sync_copy(vpn_hbm.at[pl.ds(cid * 2, 2)], vpn_v)
    plsc.subcore_barrier()
    vps = tuple(vpn_v[0, pl.ds(j * 16, 16)] for j in range(H // 16))
    vns = tuple(vpn_v[1, pl.ds(j * 16, 16)] for j in range(H // 16))

    def issue_idx(i, b):
        base = ebase + i * (2 * C)
        pltpu.async_copy(ed_hbm.at[pl.ds(base, 2 * C)], ed[b], sem_i[b])
        pltpu.async_copy(ef_hbm.at[pl.ds(efbase + i * C, C)], efv[b], sem_i[b])

    def wait_idx(b):
        pltpu.make_async_copy(ed_hbm.at[pl.ds(0, 2 * C)], ed[b], sem_i[b]).wait()
        pltpu.make_async_copy(ef_hbm.at[pl.ds(0, C)], efv[b], sem_i[b]).wait()

    def adjust_idx(b):
        edb = ed[b]
        for q in range(CH // 16):
            sl = pl.ds(q * 16, 16)
            sh = pl.ds(CH + q * 16, 16)
            sga[b][sl] = edb[sl] + goff
            sgb[b][sl] = edb[sh] + goff
        for q in range(CH // 16):
            sl = pl.ds(q * 16, 16)
            sh = pl.ds(CH + q * 16, 16)
            dga[b][sl] = edb[pl.ds(C + q * 16, 16)] + goff
            dgb[b][sl] = edb[pl.ds(C + CH + q * 16, 16)] + goff
            dsa[b][sl] = edb[pl.ds(C + q * 16, 16)]
            dsb[b][sl] = edb[pl.ds(C + CH + q * 16, 16)]

    def issue_gathers(b):
        pltpu.async_copy(a_hbm.at[sga[b]], ar[b].at[pl.ds(0, CH)], sem_a[b])
        pltpu.async_copy(a_hbm.at[sgb[b]], ar[b].at[pl.ds(CH, CH)], sem_a[b])
        pltpu.async_copy(b_hbm.at[dga[b]], br[b].at[pl.ds(0, CH)], sem_b[b])
        pltpu.async_copy(b_hbm.at[dgb[b]], br[b].at[pl.ds(CH, CH)], sem_b[b])

    def wait_gathers(b):
        pltpu.make_async_copy(a_hbm.at[sga[b]], ar[b].at[pl.ds(0, CH)], sem_a[b]).wait()
        pltpu.make_async_copy(a_hbm.at[sgb[b]], ar[b].at[pl.ds(CH, CH)], sem_a[b]).wait()
        pltpu.make_async_copy(b_hbm.at[dga[b]], br[b].at[pl.ds(0, CH)], sem_b[b]).wait()
        pltpu.make_async_copy(b_hbm.at[dgb[b]], br[b].at[pl.ds(CH, CH)], sem_b[b]).wait()

    def issue_scatter(b):
        pltpu.async_copy(mg[b].at[pl.ds(0, CH)], agg_sp.at[dsa[b]], sem_s[b],
                         add=True)
        pltpu.async_copy(mg[b].at[pl.ds(CH, CH)], agg_sp.at[dsb[b]], sem_s[b],
                         add=True)

    def wait_scatter(b):
        pltpu.make_async_copy(mg[b].at[pl.ds(0, CH)], agg_sp.at[dsa[b]],
                              sem_s[b]).wait()
        pltpu.make_async_copy(mg[b].at[pl.ds(CH, CH)], agg_sp.at[dsb[b]],
                              sem_s[b]).wait()

    def compute(b):
        arb, brb, efb, mgb = ar[b], br[b], efv[b], mg[b]

        def grp(q):
            ev = efb[pl.ds(q * 16, 16)]
            spv = jnp.maximum(ev, 0.0)
            snv = jnp.maximum(-ev, 0.0)

            def edge(rr):
                lane = jnp.full((16,), 0, jnp.int32) + rr
                sp = spv.at[lane].get(mode="promise_in_bounds")
                sn = snv.at[lane].get(mode="promise_in_bounds")
                e = q * 16 + rr
                for j in range(H // 16):
                    sl = pl.ds(j * 16, 16)
                    v = arb[e, sl] + brb[e, sl] + sp * vps[j] + sn * vns[j]
                    mgb[e, sl] = jnp.maximum(v, 0.0)

            plsc.parallel_loop(0, 16, 1, unroll=2)(edge)

        plsc.parallel_loop(0, C // 16, 1, unroll=1)(grp)

    def body(i, b):
        o = 1 - b

        @pl.when(i >= 1)
        def _():
            wait_scatter(o)

        @pl.when(i + 1 < NCH)
        def _():
            wait_idx(o)
            adjust_idx(o)
            issue_gathers(o)

        wait_gathers(b)
        compute(b)
        issue_scatter(b)

        @pl.when(i + 2 < NCH)
        def _():
            issue_idx(i + 2, b)

    issue_idx(0, 0)
    issue_idx(1, 1)
    wait_idx(0)
    adjust_idx(0)
    issue_gathers(0)

    def pair(t, carry):
        body(2 * t, 0)
        body(2 * t + 1, 1)
        return carry

    lax.fori_loop(0, NCH // 2, pair, 0)
    body(jnp.int32(NCH - 1), 0)
    wait_scatter(0)
    plsc.subcore_barrier()

    rows = agg_sp.at[pl.ds(sid * RPT, RPT)]

    @pl.when(cid == 0)
    def _():
        pltpu.sync_copy(rows, out0_hbm.at[pl.ds(sid * RPT, RPT)])

    @pl.when(cid == 1)
    def _():
        pltpu.sync_copy(rows, out1_hbm.at[pl.ds(sid * RPT, RPT)])


_edge_call = functools.partial(
    pl.kernel,
    out_type=(
        jax.ShapeDtypeStruct((NP, H), jnp.float32),
        jax.ShapeDtypeStruct((NP, H), jnp.float32),
    ),
    mesh=plsc.VectorSubcoreMesh(
        core_axis_name="c", subcore_axis_name="s",
        num_cores=NC, num_subcores=NS),
    compiler_params=pltpu.CompilerParams(use_tc_tiling_on_sc=False),
    scratch_types=(
        [pltpu.VMEM_SHARED((NP, H), jnp.float32)]
        + [pltpu.VMEM((2 * C,), jnp.int32)] * 2
        + [pltpu.VMEM((C,), jnp.float32)] * 2
        + [pltpu.VMEM((CH,), jnp.int32)] * 12
        + [pltpu.VMEM((C, H), jnp.float32)] * 6
        + [pltpu.VMEM((2, H), jnp.float32)]
        + [pltpu.SemaphoreType.DMA] * 9
    ),
)(_edge_body)


def _dec_body(ne_ref, g0_ref, g1_ref, wu1_ref, wu2a_ref, wu2b_ref,
              wd1a_ref, wd1b_ref, wd2_ref, out_ref):
    ne = ne_ref[...]
    lo = jnp.maximum(
        jnp.dot(ne, wu1_ref[...], preferred_element_type=jnp.float32)
        + jnp.dot(g0_ref[...], wu2a_ref[...], preferred_element_type=jnp.float32)
        + jnp.dot(g1_ref[...], wu2b_ref[...], preferred_element_type=jnp.float32),
        0.0)
    h = jnp.maximum(
        jnp.dot(ne, wd1a_ref[...], preferred_element_type=jnp.float32)
        + jnp.dot(lo, wd1b_ref[...], preferred_element_type=jnp.float32), 0.0)
    out_ref[...] = jnp.dot(h, wd2_ref[...], preferred_element_type=jnp.float32)


def _decode(ne, g0, g1, wu1, wu2a, wu2b, wd1a, wd1b, wd2p):
    row = pl.BlockSpec((RB, L), lambda i: (i, 0))
    half = pl.BlockSpec((RB, H), lambda i: (i, 0))
    wL = pl.BlockSpec((L, L), lambda i: (0, 0))
    wH = pl.BlockSpec((H, L), lambda i: (0, 0))
    return pl.pallas_call(
        _dec_body,
        grid=(NSTEPS,),
        in_specs=[row, half, half, wL, wH, wH, wL, wL, wL],
        out_specs=row,
        out_shape=jax.ShapeDtypeStruct((N, L), jnp.float32),
    )(ne, g0, g1, wu1, wu2a, wu2b, wd1a, wd1b, wd2p)


def kernel(node_features, edge_features, latent_features, edge_index,
           W_node, W_edge, W_msg, W_upd, W_dec1, W_dec2):
    nf_b = jnp.broadcast_to(
        node_features.astype(jnp.float32)[:, None], (N, L))
    lat = latent_features.astype(jnp.float32)
    ne, a, b, vpn = _encode(
        nf_b, lat, W_node[0:1], W_node[1:], W_edge,
        W_msg[0:L], W_msg[L:2 * L], W_msg[2 * L:])
    a2 = jnp.concatenate([a[:, :H], a[:, H:]], axis=0)
    b2 = jnp.concatenate([b[:, :H], b[:, H:]], axis=0)
    vpnr = jnp.stack([vpn[0, :H], vpn[1, :H], vpn[0, H:], vpn[1, H:]])
    src = edge_index[0].astype(jnp.int32).reshape(NS, NCH, C)
    dst = edge_index[1].astype(jnp.int32).reshape(NS, NCH, C)
    edata = jnp.stack([src, dst], axis=2).reshape(-1)
    ef = edge_features.astype(jnp.float32)
    zer = jnp.zeros((RPT, H), jnp.float32)
    g0, g1 = _edge_call(a2, b2, edata, ef, vpnr, zer)
    wd2p = jnp.pad(W_dec2, ((0, 0), (0, L - 1)))
    outp = _decode(ne, g0, g1, W_upd[:L], W_upd[L:L + H], W_upd[L + H:],
                   W_dec1[:L], W_dec1[L:], wd2p)
    return outp[:, :1]


# speedup vs baseline: 2.4409x; 1.0739x over previous
import functools

import jax
import jax.numpy as jnp
from jax import lax
from jax.experimental import pallas as pl
from jax.experimental.pallas import tpu as pltpu
from jax.experimental.pallas import tpu_sc as plsc

N = 10000
E = 320000
L = 128
H = 64

RB = 1000
NSTEPS = N // RB

NC = 2
NS = 16
EPT = E // NS
C = 160
CH = 80
NCH = EPT // C
NP = 10240
RPT = NP // NS


def _enc_body(nfc_ref, lat_ref, wn0_ref, wn1_ref, we_ref, wm1_ref, wm2_ref,
              wm3_ref, ne_ref, a_ref, b_ref, vpn_ref):
    ne = jnp.maximum(
        nfc_ref[...] * wn0_ref[...]
        + jnp.dot(lat_ref[...], wn1_ref[...], preferred_element_type=jnp.float32),
        0.0)
    ne_ref[...] = ne
    av = jnp.dot(ne, wm1_ref[...], preferred_element_type=jnp.float32)
    bv = jnp.dot(ne, wm2_ref[...], preferred_element_type=jnp.float32)
    a_ref[0] = av[:, :H]
    a_ref[1] = av[:, H:]
    b_ref[0] = bv[:, :H]
    b_ref[1] = bv[:, H:]
    ep = jnp.maximum(we_ref[...], 0.0)
    en = jnp.maximum(-we_ref[...], 0.0)
    vp = jnp.dot(ep, wm3_ref[...], preferred_element_type=jnp.float32)
    vn = jnp.dot(en, wm3_ref[...], preferred_element_type=jnp.float32)
    vpn_ref[...] = jnp.concatenate([vp, vn], axis=0)


def _encode(nf_c, lat, wn0, wn1, we, wm1, wm2, wm3):
    row = pl.BlockSpec((RB, L), lambda i: (i, 0))
    col = pl.BlockSpec((RB, 1), lambda i: (i, 0))
    w1 = pl.BlockSpec((1, L), lambda i: (0, 0))
    wL = pl.BlockSpec((L, L), lambda i: (0, 0))
    halves = pl.BlockSpec((2, RB, H), lambda i: (0, i, 0))
    return pl.pallas_call(
        _enc_body,
        grid=(NSTEPS,),
        in_specs=[col, row, w1, wL, w1, wL, wL, wL],
        out_specs=[row, halves, halves, pl.BlockSpec((2, L), lambda i: (0, 0))],
        out_shape=[
            jax.ShapeDtypeStruct((N, L), jnp.float32),
            jax.ShapeDtypeStruct((2, N, H), jnp.float32),
            jax.ShapeDtypeStruct((2, N, H), jnp.float32),
            jax.ShapeDtypeStruct((2, L), jnp.float32),
        ],
    )(nf_c, lat, wn0, wn1, we, wm1, wm2, wm3)


def _edge_body(a_hbm, b_hbm, ed_hbm, ef_hbm, vpn_hbm, zer_hbm,
               out0_hbm, out1_hbm,
               agg_sp,
               ed0, ed1, ef0, ef1,
               sga0, sga1, sgb0, sgb1, dga0, dga1, dgb0, dgb1,
               dsa0, dsa1, dsb0, dsb1,
               ar0, ar1, br0, br1, mg0, mg1, vpn_v,
               sem_i0, sem_i1, sem_a0, sem_a1, sem_b0, sem_b1,
               sem_s0, sem_s1, sem_z):
    ed = (ed0, ed1)
    efv = (ef0, ef1)
    sga = (sga0, sga1)
    sgb = (sgb0, sgb1)
    dga = (dga0, dga1)
    dgb = (dgb0, dgb1)
    dsa = (dsa0, dsa1)
    dsb = (dsb0, dsb1)
    ar = (ar0, ar1)
    br = (br0, br1)
    mg = (mg0, mg1)
    sem_i = (sem_i0, sem_i1)
    sem_a = (sem_a0, sem_a1)
    sem_b = (sem_b0, sem_b1)
    sem_s = (sem_s0, sem_s1)

    cid = lax.axis_index("c")
    sid = lax.axis_index("s")
    ebase = sid * NCH * 2 * C
    efbase = sid * EPT
    goff = cid * N

    pltpu.async_copy(zer_hbm, agg_sp.at[pl.ds(sid * RPT, RPT)], sem_z).wait()
    pltpu.sync_copy(vpn_hbm.at[pl.ds(cid * 2, 2)], vpn_v)
    plsc.subcore_barrier()
    vps = tuple(vpn_v[0, pl.ds(j * 16, 16)] for j in range(H // 16))
    vns = tuple(vpn_v[1, pl.ds(j * 16, 16)] for j in range(H // 16))

    def issue_idx(i, b):
        base = ebase + i * (2 * C)
        pltpu.async_copy(ed_hbm.at[pl.ds(base, 2 * C)], ed[b], sem_i[b])
        pltpu.async_copy(ef_hbm.at[pl.ds(efbase + i * C, C)], efv[b], sem_i[b])

    def wait_idx(b):
        pltpu.make_async_copy(ed_hbm.at[pl.ds(0, 2 * C)], ed[b], sem_i[b]).wait()
        pltpu.make_async_copy(ef_hbm.at[pl.ds(0, C)], efv[b], sem_i[b]).wait()

    def adjust_idx(b):
        edb = ed[b]
        for q in range(CH // 16):
            sl = pl.ds(q * 16, 16)
            sh = pl.ds(CH + q * 16, 16)
            sga[b][sl] = edb[sl] + goff
            sgb[b][sl] = edb[sh] + goff
        for q in range(CH // 16):
            sl = pl.ds(q * 16, 16)
            sh = pl.ds(CH + q * 16, 16)
            dga[b][sl] = edb[pl.ds(C + q * 16, 16)] + goff
            dgb[b][sl] = edb[pl.ds(C + CH + q * 16, 16)] + goff
            dsa[b][sl] = edb[pl.ds(C + q * 16, 16)]
            dsb[b][sl] = edb[pl.ds(C + CH + q * 16, 16)]

    def issue_gathers(b):
        pltpu.async_copy(a_hbm.at[sga[b]], ar[b].at[pl.ds(0, CH)], sem_a[b])
        pltpu.async_copy(a_hbm.at[sgb[b]], ar[b].at[pl.ds(CH, CH)], sem_a[b])
        pltpu.async_copy(b_hbm.at[dga[b]], br[b].at[pl.ds(0, CH)], sem_b[b])
        pltpu.async_copy(b_hbm.at[dgb[b]], br[b].at[pl.ds(CH, CH)], sem_b[b])

    def wait_gathers(b):
        pltpu.make_async_copy(a_hbm.at[sga[b]], ar[b].at[pl.ds(0, CH)], sem_a[b]).wait()
        pltpu.make_async_copy(a_hbm.at[sgb[b]], ar[b].at[pl.ds(CH, CH)], sem_a[b]).wait()
        pltpu.make_async_copy(b_hbm.at[dga[b]], br[b].at[pl.ds(0, CH)], sem_b[b]).wait()
        pltpu.make_async_copy(b_hbm.at[dgb[b]], br[b].at[pl.ds(CH, CH)], sem_b[b]).wait()

    def issue_scatter(b):
        pltpu.async_copy(mg[b].at[pl.ds(0, CH)], agg_sp.at[dsa[b]], sem_s[b],
                         add=True)
        pltpu.async_copy(mg[b].at[pl.ds(CH, CH)], agg_sp.at[dsb[b]], sem_s[b],
                         add=True)

    def wait_scatter(b):
        pltpu.make_async_copy(mg[b].at[pl.ds(0, CH)], agg_sp.at[dsa[b]],
                              sem_s[b]).wait()
        pltpu.make_async_copy(mg[b].at[pl.ds(CH, CH)], agg_sp.at[dsb[b]],
                              sem_s[b]).wait()

    def compute(b):
        arb, brb, efb, mgb = ar[b], br[b], efv[b], mg[b]

        def grp(q):
            ev = efb[pl.ds(q * 16, 16)]
            spv = jnp.maximum(ev, 0.0)
            snv = jnp.maximum(-ev, 0.0)

            def edge(rr):
                lane = jnp.full((16,), 0, jnp.int32) + rr
                sp = spv.at[lane].get(mode="promise_in_bounds")
                sn = snv.at[lane].get(mode="promise_in_bounds")
                e = q * 16 + rr
                for j in range(H // 16):
                    sl = pl.ds(j * 16, 16)
                    v = arb[e, sl] + brb[e, sl] + sp * vps[j] + sn * vns[j]
                    mgb[e, sl] = jnp.maximum(v, 0.0)

            plsc.parallel_loop(0, 16, 1, unroll=2)(edge)

        plsc.parallel_loop(0, C // 16, 1, unroll=1)(grp)

    def body(i, b):
        o = 1 - b

        @pl.when(i >= 1)
        def _():
            wait_scatter(o)

        @pl.when(i + 1 < NCH)
        def _():
            wait_idx(o)
            adjust_idx(o)
            issue_gathers(o)

        wait_gathers(b)
        compute(b)
        issue_scatter(b)

        @pl.when(i + 2 < NCH)
        def _():
            issue_idx(i + 2, b)

    issue_idx(0, 0)
    issue_idx(1, 1)
    wait_idx(0)
    adjust_idx(0)
    issue_gathers(0)

    def pair(t, carry):
        body(2 * t, 0)
        body(2 * t + 1, 1)
        return carry

    lax.fori_loop(0, NCH // 2, pair, 0)
    body(jnp.int32(NCH - 1), 0)
    wait_scatter(0)
    plsc.subcore_barrier()

    rows = agg_sp.at[pl.ds(sid * RPT, RPT)]

    @pl.when(cid == 0)
    def _():
        pltpu.sync_copy(rows, out0_hbm.at[pl.ds(sid * RPT, RPT)])

    @pl.when(cid == 1)
    def _():
        pltpu.sync_copy(rows, out1_hbm.at[pl.ds(sid * RPT, RPT)])


_edge_call = functools.partial(
    pl.kernel,
    out_type=(
        jax.ShapeDtypeStruct((NP, H), jnp.float32),
        jax.ShapeDtypeStruct((NP, H), jnp.float32),
    ),
    mesh=plsc.VectorSubcoreMesh(
        core_axis_name="c", subcore_axis_name="s",
        num_cores=NC, num_subcores=NS),
    compiler_params=pltpu.CompilerParams(use_tc_tiling_on_sc=False),
    scratch_types=(
        [pltpu.VMEM_SHARED((NP, H), jnp.float32)]
        + [pltpu.VMEM((2 * C,), jnp.int32)] * 2
        + [pltpu.VMEM((C,), jnp.float32)] * 2
        + [pltpu.VMEM((CH,), jnp.int32)] * 12
        + [pltpu.VMEM((C, H), jnp.float32)] * 6
        + [pltpu.VMEM((2, H), jnp.float32)]
        + [pltpu.SemaphoreType.DMA] * 9
    ),
)(_edge_body)


def _dec_body(ne_ref, g0_ref, g1_ref, wu1_ref, wu2a_ref, wu2b_ref,
              wd1a_ref, wd1b_ref, wd2_ref, out_ref):
    ne = ne_ref[...]
    lo = jnp.maximum(
        jnp.dot(ne, wu1_ref[...], preferred_element_type=jnp.float32)
        + jnp.dot(g0_ref[...], wu2a_ref[...], preferred_element_type=jnp.float32)
        + jnp.dot(g1_ref[...], wu2b_ref[...], preferred_element_type=jnp.float32),
        0.0)
    h = jnp.maximum(
        jnp.dot(ne, wd1a_ref[...], preferred_element_type=jnp.float32)
        + jnp.dot(lo, wd1b_ref[...], preferred_element_type=jnp.float32), 0.0)
    out_ref[...] = jnp.dot(h, wd2_ref[...], preferred_element_type=jnp.float32)


def _decode(ne, g0, g1, wu1, wu2a, wu2b, wd1a, wd1b, wd2p):
    row = pl.BlockSpec((RB, L), lambda i: (i, 0))
    half = pl.BlockSpec((RB, H), lambda i: (i, 0))
    wL = pl.BlockSpec((L, L), lambda i: (0, 0))
    wH = pl.BlockSpec((H, L), lambda i: (0, 0))
    return pl.pallas_call(
        _dec_body,
        grid=(NSTEPS,),
        in_specs=[row, half, half, wL, wH, wH, wL, wL, wL],
        out_specs=row,
        out_shape=jax.ShapeDtypeStruct((N, L), jnp.float32),
    )(ne, g0, g1, wu1, wu2a, wu2b, wd1a, wd1b, wd2p)


def kernel(node_features, edge_features, latent_features, edge_index,
           W_node, W_edge, W_msg, W_upd, W_dec1, W_dec2):
    nf_c = node_features.astype(jnp.float32)[:, None]
    lat = latent_features.astype(jnp.float32)
    ne, a3, b3, vpn = _encode(
        nf_c, lat, W_node[0:1], W_node[1:], W_edge,
        W_msg[0:L], W_msg[L:2 * L], W_msg[2 * L:])
    a2 = a3.reshape(2 * N, H)
    b2 = b3.reshape(2 * N, H)
    vpnr = jnp.stack([vpn[0, :H], vpn[1, :H], vpn[0, H:], vpn[1, H:]])
    src = edge_index[0].astype(jnp.int32).reshape(NS, NCH, C)
    dst = edge_index[1].astype(jnp.int32).reshape(NS, NCH, C)
    edata = jnp.stack([src, dst], axis=2).reshape(-1)
    ef = edge_features.astype(jnp.float32)
    zer = jnp.zeros((RPT, H), jnp.float32)
    g0, g1 = _edge_call(a2, b2, edata, ef, vpnr, zer)
    wd2p = jnp.pad(W_dec2, ((0, 0), (0, L - 1)))
    outp = _decode(ne, g0, g1, W_upd[:L], W_upd[L:L + H], W_upd[L + H:],
                   W_dec1[:L], W_dec1[L:], wd2p)
    return outp[:, :1]


# speedup vs baseline: 2.4942x; 1.0218x over previous
import functools

import jax
import jax.numpy as jnp
from jax import lax
from jax.experimental import pallas as pl
from jax.experimental.pallas import tpu as pltpu
from jax.experimental.pallas import tpu_sc as plsc

N = 10000
E = 320000
L = 128
H = 64

RB = 1000
NSTEPS = N // RB

NC = 2
NS = 16
EPT = E // NS
C = 160
CH = 80
NCH = EPT // C
NP = 10240
RPT = NP // NS


def _enc_body(nfc_ref, lat_ref, wn0_ref, wn1_ref, we_ref, wm1_ref, wm2_ref,
              wm3_ref, ne_ref, a_ref, b_ref, vpn_ref):
    ne = jnp.maximum(
        nfc_ref[...] * wn0_ref[...]
        + jnp.dot(lat_ref[...], wn1_ref[...], preferred_element_type=jnp.float32),
        0.0)
    ne_ref[...] = ne
    av = jnp.dot(ne, wm1_ref[...], preferred_element_type=jnp.float32)
    bv = jnp.dot(ne, wm2_ref[...], preferred_element_type=jnp.float32)
    a_ref[0] = av[:, :H]
    a_ref[1] = av[:, H:]
    b_ref[0] = bv[:, :H]
    b_ref[1] = bv[:, H:]
    ep = jnp.maximum(we_ref[...], 0.0)
    en = jnp.maximum(-we_ref[...], 0.0)
    vp = jnp.dot(ep, wm3_ref[...], preferred_element_type=jnp.float32)
    vn = jnp.dot(en, wm3_ref[...], preferred_element_type=jnp.float32)
    vpn_ref[...] = jnp.concatenate([vp, vn], axis=0)


def _encode(nf_c, lat, wn0, wn1, we, wm1, wm2, wm3):
    row = pl.BlockSpec((RB, L), lambda i: (i, 0))
    col = pl.BlockSpec((RB, 1), lambda i: (i, 0))
    w1 = pl.BlockSpec((1, L), lambda i: (0, 0))
    wL = pl.BlockSpec((L, L), lambda i: (0, 0))
    halves = pl.BlockSpec((2, RB, H), lambda i: (0, i, 0))
    return pl.pallas_call(
        _enc_body,
        grid=(NSTEPS,),
        in_specs=[col, row, w1, wL, w1, wL, wL, wL],
        out_specs=[row, halves, halves, pl.BlockSpec((2, L), lambda i: (0, 0))],
        out_shape=[
            jax.ShapeDtypeStruct((N, L), jnp.float32),
            jax.ShapeDtypeStruct((2, N, H), jnp.float32),
            jax.ShapeDtypeStruct((2, N, H), jnp.float32),
            jax.ShapeDtypeStruct((2, L), jnp.float32),
        ],
    )(nf_c, lat, wn0, wn1, we, wm1, wm2, wm3)


def _edge_body(a_hbm, b_hbm, ed_hbm, ef_hbm, vpn_hbm, zer_hbm,
               out0_hbm, out1_hbm,
               agg_sp,
               ed0, ed1, ef0, ef1,
               sga0, sga1, sgb0, sgb1, dga0, dga1, dgb0, dgb1,
               dsa0, dsa1, dsb0, dsb1,
               ar0, ar1, br0, br1, mg0, mg1, vpn_v,
               sem_i0, sem_i1, sem_a0, sem_a1, sem_b0, sem_b1,
               sem_s0, sem_s1, sem_z):
    ed = (ed0, ed1)
    efv = (ef0, ef1)
    sga = (sga0, sga1)
    sgb = (sgb0, sgb1)
    dga = (dga0, dga1)
    dgb = (dgb0, dgb1)
    dsa = (dsa0, dsa1)
    dsb = (dsb0, dsb1)
    ar = (ar0, ar1)
    br = (br0, br1)
    mg = (mg0, mg1)
    sem_i = (sem_i0, sem_i1)
    sem_a = (sem_a0, sem_a1)
    sem_b = (sem_b0, sem_b1)
    sem_s = (sem_s0, sem_s1)

    cid = lax.axis_index("c")
    sid = lax.axis_index("s")
    ebase = sid * NCH * 2 * C
    efbase = sid * EPT
    goff = cid * N

    pltpu.async_copy(zer_hbm, agg_sp.at[pl.ds(sid * RPT, RPT)], sem_z).wait()
    pltpu.sync_copy(vpn_hbm.at[pl.ds(cid * 2, 2)], vpn_v)
    plsc.subcore_barrier()
    vps = tuple(vpn_v[0, pl.ds(j * 16, 16)] for j in range(H // 16))
    vns = tuple(vpn_v[1, pl.ds(j * 16, 16)] for j in range(H // 16))

    def issue_idx(i, b):
        base = ebase + i * (2 * C)
        pltpu.async_copy(ed_hbm.at[pl.ds(base, 2 * C)], ed[b], sem_i[b])
        pltpu.async_copy(ef_hbm.at[pl.ds(efbase + i * C, C)], efv[b], sem_i[b])

    def wait_idx(b):
        pltpu.make_async_copy(ed_hbm.at[pl.ds(0, 2 * C)], ed[b], sem_i[b]).wait()
        pltpu.make_async_copy(ef_hbm.at[pl.ds(0, C)], efv[b], sem_i[b]).wait()

    def adjust_idx(b):
        edb = ed[b]
        for q in range(CH // 16):
            sl = pl.ds(q * 16, 16)
            sh = pl.ds(CH + q * 16, 16)
            sga[b][sl] = edb[sl] + goff
            sgb[b][sl] = edb[sh] + goff
        for q in range(CH // 16):
            sl = pl.ds(q * 16, 16)
            sh = pl.ds(CH + q * 16, 16)
            dga[b][sl] = edb[pl.ds(C + q * 16, 16)] + goff
            dgb[b][sl] = edb[pl.ds(C + CH + q * 16, 16)] + goff
            dsa[b][sl] = edb[pl.ds(C + q * 16, 16)]
            dsb[b][sl] = edb[pl.ds(C + CH + q * 16, 16)]

    def issue_gathers(b):
        pltpu.async_copy(a_hbm.at[sga[b]], ar[b].at[pl.ds(0, CH)], sem_a[b])
        pltpu.async_copy(a_hbm.at[sgb[b]], ar[b].at[pl.ds(CH, CH)], sem_a[b])
        pltpu.async_copy(b_hbm.at[dga[b]], br[b].at[pl.ds(0, CH)], sem_b[b])
        pltpu.async_copy(b_hbm.at[dgb[b]], br[b].at[pl.ds(CH, CH)], sem_b[b])

    def wait_gathers(b):
        pltpu.make_async_copy(a_hbm.at[sga[b]], ar[b].at[pl.ds(0, CH)], sem_a[b]).wait()
        pltpu.make_async_copy(a_hbm.at[sgb[b]], ar[b].at[pl.ds(CH, CH)], sem_a[b]).wait()
        pltpu.make_async_copy(b_hbm.at[dga[b]], br[b].at[pl.ds(0, CH)], sem_b[b]).wait()
        pltpu.make_async_copy(b_hbm.at[dgb[b]], br[b].at[pl.ds(CH, CH)], sem_b[b]).wait()

    def issue_scatter(b):
        pltpu.async_copy(mg[b].at[pl.ds(0, CH)], agg_sp.at[dsa[b]], sem_s[b],
                         add=True)
        pltpu.async_copy(mg[b].at[pl.ds(CH, CH)], agg_sp.at[dsb[b]], sem_s[b],
                         add=True)

    def wait_scatter(b):
        pltpu.make_async_copy(mg[b].at[pl.ds(0, CH)], agg_sp.at[dsa[b]],
                              sem_s[b]).wait()
        pltpu.make_async_copy(mg[b].at[pl.ds(CH, CH)], agg_sp.at[dsb[b]],
                              sem_s[b]).wait()

    def compute(b):
        arb, brb, efb, mgb = ar[b], br[b], efv[b], mg[b]

        def grp(q):
            ev = efb[pl.ds(q * 16, 16)]
            spv = jnp.maximum(ev, 0.0)
            snv = jnp.maximum(-ev, 0.0)

            def edge(rr):
                lane = jnp.full((16,), 0, jnp.int32) + rr
                sp = spv.at[lane].get(mode="promise_in_bounds")
                sn = snv.at[lane].get(mode="promise_in_bounds")
                e = q * 16 + rr
                for j in range(H // 16):
                    sl = pl.ds(j * 16, 16)
                    v = arb[e, sl] + brb[e, sl] + sp * vps[j] + sn * vns[j]
                    mgb[e, sl] = jnp.maximum(v, 0.0)

            plsc.parallel_loop(0, 16, 1, unroll=4)(edge)

        plsc.parallel_loop(0, C // 16, 1, unroll=1)(grp)

    def body(i, b):
        o = 1 - b

        @pl.when(i >= 1)
        def _():
            wait_scatter(o)

        @pl.when(i + 1 < NCH)
        def _():
            wait_idx(o)
            adjust_idx(o)
            issue_gathers(o)

        wait_gathers(b)
        compute(b)
        issue_scatter(b)

        @pl.when(i + 2 < NCH)
        def _():
            issue_idx(i + 2, b)

    issue_idx(0, 0)
    issue_idx(1, 1)
    wait_idx(0)
    adjust_idx(0)
    issue_gathers(0)

    def pair(t, carry):
        body(2 * t, 0)
        body(2 * t + 1, 1)
        return carry

    lax.fori_loop(0, NCH // 2, pair, 0)
    body(jnp.int32(NCH - 1), 0)
    wait_scatter(0)
    plsc.subcore_barrier()

    rows = agg_sp.at[pl.ds(sid * RPT, RPT)]

    @pl.when(cid == 0)
    def _():
        pltpu.sync_copy(rows, out0_hbm.at[pl.ds(sid * RPT, RPT)])

    @pl.when(cid == 1)
    def _():
        pltpu.sync_copy(rows, out1_hbm.at[pl.ds(sid * RPT, RPT)])


_edge_call = functools.partial(
    pl.kernel,
    out_type=(
        jax.ShapeDtypeStruct((NP, H), jnp.float32),
        jax.ShapeDtypeStruct((NP, H), jnp.float32),
    ),
    mesh=plsc.VectorSubcoreMesh(
        core_axis_name="c", subcore_axis_name="s",
        num_cores=NC, num_subcores=NS),
    compiler_params=pltpu.CompilerParams(use_tc_tiling_on_sc=False),
    scratch_types=(
        [pltpu.VMEM_SHARED((NP, H), jnp.float32)]
        + [pltpu.VMEM((2 * C,), jnp.int32)] * 2
        + [pltpu.VMEM((C,), jnp.float32)] * 2
        + [pltpu.VMEM((CH,), jnp.int32)] * 12
        + [pltpu.VMEM((C, H), jnp.float32)] * 6
        + [pltpu.VMEM((2, H), jnp.float32)]
        + [pltpu.SemaphoreType.DMA] * 9
    ),
)(_edge_body)


def _dec_body(ne_ref, g0_ref, g1_ref, wu1_ref, wu2a_ref, wu2b_ref,
              wd1a_ref, wd1b_ref, wd2_ref, out_ref):
    ne = ne_ref[...]
    lo = jnp.maximum(
        jnp.dot(ne, wu1_ref[...], preferred_element_type=jnp.float32)
        + jnp.dot(g0_ref[...], wu2a_ref[...], preferred_element_type=jnp.float32)
        + jnp.dot(g1_ref[...], wu2b_ref[...], preferred_element_type=jnp.float32),
        0.0)
    h = jnp.maximum(
        jnp.dot(ne, wd1a_ref[...], preferred_element_type=jnp.float32)
        + jnp.dot(lo, wd1b_ref[...], preferred_element_type=jnp.float32), 0.0)
    out_ref[...] = jnp.dot(h, wd2_ref[...], preferred_element_type=jnp.float32)


def _decode(ne, g0, g1, wu1, wu2a, wu2b, wd1a, wd1b, wd2p):
    row = pl.BlockSpec((RB, L), lambda i: (i, 0))
    half = pl.BlockSpec((RB, H), lambda i: (i, 0))
    wL = pl.BlockSpec((L, L), lambda i: (0, 0))
    wH = pl.BlockSpec((H, L), lambda i: (0, 0))
    return pl.pallas_call(
        _dec_body,
        grid=(NSTEPS,),
        in_specs=[row, half, half, wL, wH, wH, wL, wL, wL],
        out_specs=row,
        out_shape=jax.ShapeDtypeStruct((N, L), jnp.float32),
    )(ne, g0, g1, wu1, wu2a, wu2b, wd1a, wd1b, wd2p)


def kernel(node_features, edge_features, latent_features, edge_index,
           W_node, W_edge, W_msg, W_upd, W_dec1, W_dec2):
    nf_c = node_features.astype(jnp.float32)[:, None]
    lat = latent_features.astype(jnp.float32)
    ne, a3, b3, vpn = _encode(
        nf_c, lat, W_node[0:1], W_node[1:], W_edge,
        W_msg[0:L], W_msg[L:2 * L], W_msg[2 * L:])
    a2 = a3.reshape(2 * N, H)
    b2 = b3.reshape(2 * N, H)
    vpnr = jnp.stack([vpn[0, :H], vpn[1, :H], vpn[0, H:], vpn[1, H:]])
    src = edge_index[0].astype(jnp.int32).reshape(NS, NCH, C)
    dst = edge_index[1].astype(jnp.int32).reshape(NS, NCH, C)
    edata = jnp.stack([src, dst], axis=2).reshape(-1)
    ef = edge_features.astype(jnp.float32)
    zer = jnp.zeros((RPT, H), jnp.float32)
    g0, g1 = _edge_call(a2, b2, edata, ef, vpnr, zer)
    wd2p = jnp.pad(W_dec2, ((0, 0), (0, L - 1)))
    outp = _decode(ne, g0, g1, W_upd[:L], W_upd[L:L + H], W_upd[L + H:],
                   W_dec1[:L], W_dec1[L:], wd2p)
    return outp[:, :1]


# speedup vs baseline: 2.5950x; 1.0404x over previous
import functools

import jax
import jax.numpy as jnp
from jax import lax
from jax.experimental import pallas as pl
from jax.experimental.pallas import tpu as pltpu
from jax.experimental.pallas import tpu_sc as plsc

N = 10000
E = 320000
L = 128
H = 64

RB = 1000
NSTEPS = N // RB

NC = 2
NS = 16
EPT = E // NS
C = 160
CH = 80
NCH = EPT // C
NP = 10240
RPT = NP // NS


def _enc_body(nfc_ref, lat_ref, wn0_ref, wn1_ref, we_ref, wm1_ref, wm2_ref,
              wm3_ref, ne_ref, a_ref, b_ref, vpn_ref):
    ne = jnp.maximum(
        nfc_ref[...] * wn0_ref[...]
        + jnp.dot(lat_ref[...], wn1_ref[...], preferred_element_type=jnp.float32),
        0.0)
    ne_ref[...] = ne
    av = jnp.dot(ne, wm1_ref[...], preferred_element_type=jnp.float32)
    bv = jnp.dot(ne, wm2_ref[...], preferred_element_type=jnp.float32)
    a_ref[0] = av[:, :H]
    a_ref[1] = av[:, H:]
    b_ref[0] = bv[:, :H]
    b_ref[1] = bv[:, H:]
    ep = jnp.maximum(we_ref[...], 0.0)
    en = jnp.maximum(-we_ref[...], 0.0)
    vp = jnp.dot(ep, wm3_ref[...], preferred_element_type=jnp.float32)
    vn = jnp.dot(en, wm3_ref[...], preferred_element_type=jnp.float32)
    vpn_ref[...] = jnp.concatenate([vp, vn], axis=0)


def _encode(nf_c, lat, wn0, wn1, we, wm1, wm2, wm3):
    row = pl.BlockSpec((RB, L), lambda i: (i, 0))
    col = pl.BlockSpec((RB, 1), lambda i: (i, 0))
    w1 = pl.BlockSpec((1, L), lambda i: (0, 0))
    wL = pl.BlockSpec((L, L), lambda i: (0, 0))
    halves = pl.BlockSpec((2, RB, H), lambda i: (0, i, 0))
    return pl.pallas_call(
        _enc_body,
        grid=(NSTEPS,),
        in_specs=[col, row, w1, wL, w1, wL, wL, wL],
        out_specs=[row, halves, halves, pl.BlockSpec((2, L), lambda i: (0, 0))],
        out_shape=[
            jax.ShapeDtypeStruct((N, L), jnp.float32),
            jax.ShapeDtypeStruct((2, N, H), jnp.float32),
            jax.ShapeDtypeStruct((2, N, H), jnp.float32),
            jax.ShapeDtypeStruct((2, L), jnp.float32),
        ],
    )(nf_c, lat, wn0, wn1, we, wm1, wm2, wm3)


def _edge_body(a_hbm, b_hbm, ed_hbm, ef_hbm, vpn_hbm, zer_hbm,
               out0_hbm, out1_hbm,
               agg_sp,
               ed0, ed1, ef0, ef1,
               sga0, sga1, sgb0, sgb1, dga0, dga1, dgb0, dgb1,
               dsa0, dsa1, dsb0, dsb1,
               ar0, ar1, br0, br1, mg0, mg1, vpn_v,
               sem_i0, sem_i1, sem_a0, sem_a1, sem_b0, sem_b1,
               sem_s0, sem_s1, sem_z):
    ed = (ed0, ed1)
    efv = (ef0, ef1)
    sga = (sga0, sga1)
    sgb = (sgb0, sgb1)
    dga = (dga0, dga1)
    dgb = (dgb0, dgb1)
    dsa = (dsa0, dsa1)
    dsb = (dsb0, dsb1)
    ar = (ar0, ar1)
    br = (br0, br1)
    mg = (mg0, mg1)
    sem_i = (sem_i0, sem_i1)
    sem_a = (sem_a0, sem_a1)
    sem_b = (sem_b0, sem_b1)
    sem_s = (sem_s0, sem_s1)

    cid = lax.axis_index("c")
    sid = lax.axis_index("s")
    ebase = sid * NCH * 2 * C
    efbase = sid * EPT
    goff = cid * N

    pltpu.async_copy(zer_hbm, agg_sp.at[pl.ds(sid * RPT, RPT)], sem_z).wait()
    pltpu.sync_copy(vpn_hbm.at[pl.ds(cid * 2, 2)], vpn_v)
    plsc.subcore_barrier()
    vps = tuple(vpn_v[0, pl.ds(j * 16, 16)] for j in range(H // 16))
    vns = tuple(vpn_v[1, pl.ds(j * 16, 16)] for j in range(H // 16))

    def issue_idx(i, b):
        base = ebase + i * (2 * C)
        pltpu.async_copy(ed_hbm.at[pl.ds(base, 2 * C)], ed[b], sem_i[b])
        pltpu.async_copy(ef_hbm.at[pl.ds(efbase + i * C, C)], efv[b], sem_i[b])

    def wait_idx(b):
        pltpu.make_async_copy(ed_hbm.at[pl.ds(0, 2 * C)], ed[b], sem_i[b]).wait()
        pltpu.make_async_copy(ef_hbm.at[pl.ds(0, C)], efv[b], sem_i[b]).wait()

    def adjust_idx(b):
        edb = ed[b]
        for q in range(CH // 16):
            sl = pl.ds(q * 16, 16)
            sh = pl.ds(CH + q * 16, 16)
            sga[b][sl] = edb[sl] + goff
            sgb[b][sl] = edb[sh] + goff
        for q in range(CH // 16):
            sl = pl.ds(q * 16, 16)
            sh = pl.ds(CH + q * 16, 16)
            dga[b][sl] = edb[pl.ds(C + q * 16, 16)] + goff
            dgb[b][sl] = edb[pl.ds(C + CH + q * 16, 16)] + goff
            dsa[b][sl] = edb[pl.ds(C + q * 16, 16)]
            dsb[b][sl] = edb[pl.ds(C + CH + q * 16, 16)]

    def issue_gathers(b):
        pltpu.async_copy(a_hbm.at[sga[b]], ar[b].at[pl.ds(0, CH)], sem_a[b])
        pltpu.async_copy(a_hbm.at[sgb[b]], ar[b].at[pl.ds(CH, CH)], sem_a[b])
        pltpu.async_copy(b_hbm.at[dga[b]], br[b].at[pl.ds(0, CH)], sem_b[b])
        pltpu.async_copy(b_hbm.at[dgb[b]], br[b].at[pl.ds(CH, CH)], sem_b[b])

    def wait_gathers(b):
        pltpu.make_async_copy(a_hbm.at[sga[b]], ar[b].at[pl.ds(0, CH)], sem_a[b]).wait()
        pltpu.make_async_copy(a_hbm.at[sgb[b]], ar[b].at[pl.ds(CH, CH)], sem_a[b]).wait()
        pltpu.make_async_copy(b_hbm.at[dga[b]], br[b].at[pl.ds(0, CH)], sem_b[b]).wait()
        pltpu.make_async_copy(b_hbm.at[dgb[b]], br[b].at[pl.ds(CH, CH)], sem_b[b]).wait()

    def issue_scatter(b):
        pltpu.async_copy(mg[b].at[pl.ds(0, CH)], agg_sp.at[dsa[b]], sem_s[b],
                         add=True)
        pltpu.async_copy(mg[b].at[pl.ds(CH, CH)], agg_sp.at[dsb[b]], sem_s[b],
                         add=True)

    def wait_scatter(b):
        pltpu.make_async_copy(mg[b].at[pl.ds(0, CH)], agg_sp.at[dsa[b]],
                              sem_s[b]).wait()
        pltpu.make_async_copy(mg[b].at[pl.ds(CH, CH)], agg_sp.at[dsb[b]],
                              sem_s[b]).wait()

    def compute(b):
        arb, brb, efb, mgb = ar[b], br[b], efv[b], mg[b]

        def edge(e):
            ev = efb[pl.ds(e & ~15, 16)]
            lane = jnp.full((16,), 0, jnp.int32) + (e & 15)
            evb = ev.at[lane].get(mode="promise_in_bounds")
            sp = jnp.maximum(evb, 0.0)
            sn = jnp.maximum(-evb, 0.0)
            for j in range(H // 16):
                sl = pl.ds(j * 16, 16)
                v = arb[e, sl] + brb[e, sl] + sp * vps[j] + sn * vns[j]
                mgb[e, sl] = jnp.maximum(v, 0.0)

        plsc.parallel_loop(0, C, 1, unroll=4)(edge)

    def body(i, b):
        o = 1 - b

        @pl.when(i >= 1)
        def _():
            wait_scatter(o)

        @pl.when(i + 1 < NCH)
        def _():
            wait_idx(o)
            adjust_idx(o)
            issue_gathers(o)

        wait_gathers(b)
        compute(b)
        issue_scatter(b)

        @pl.when(i + 2 < NCH)
        def _():
            issue_idx(i + 2, b)

    issue_idx(0, 0)
    issue_idx(1, 1)
    wait_idx(0)
    adjust_idx(0)
    issue_gathers(0)

    def pair(t, carry):
        body(2 * t, 0)
        body(2 * t + 1, 1)
        return carry

    lax.fori_loop(0, NCH // 2, pair, 0)
    body(jnp.int32(NCH - 1), 0)
    wait_scatter(0)
    plsc.subcore_barrier()

    rows = agg_sp.at[pl.ds(sid * RPT, RPT)]

    @pl.when(cid == 0)
    def _():
        pltpu.sync_copy(rows, out0_hbm.at[pl.ds(sid * RPT, RPT)])

    @pl.when(cid == 1)
    def _():
        pltpu.sync_copy(rows, out1_hbm.at[pl.ds(sid * RPT, RPT)])


_edge_call = functools.partial(
    pl.kernel,
    out_type=(
        jax.ShapeDtypeStruct((NP, H), jnp.float32),
        jax.ShapeDtypeStruct((NP, H), jnp.float32),
    ),
    mesh=plsc.VectorSubcoreMesh(
        core_axis_name="c", subcore_axis_name="s",
        num_cores=NC, num_subcores=NS),
    compiler_params=pltpu.CompilerParams(use_tc_tiling_on_sc=False),
    scratch_types=(
        [pltpu.VMEM_SHARED((NP, H), jnp.float32)]
        + [pltpu.VMEM((2 * C,), jnp.int32)] * 2
        + [pltpu.VMEM((C,), jnp.float32)] * 2
        + [pltpu.VMEM((CH,), jnp.int32)] * 12
        + [pltpu.VMEM((C, H), jnp.float32)] * 6
        + [pltpu.VMEM((2, H), jnp.float32)]
        + [pltpu.SemaphoreType.DMA] * 9
    ),
)(_edge_body)


def _dec_body(ne_ref, g0_ref, g1_ref, wu1_ref, wu2a_ref, wu2b_ref,
              wd1a_ref, wd1b_ref, wd2_ref, out_ref):
    ne = ne_ref[...]
    lo = jnp.maximum(
        jnp.dot(ne, wu1_ref[...], preferred_element_type=jnp.float32)
        + jnp.dot(g0_ref[...], wu2a_ref[...], preferred_element_type=jnp.float32)
        + jnp.dot(g1_ref[...], wu2b_ref[...], preferred_element_type=jnp.float32),
        0.0)
    h = jnp.maximum(
        jnp.dot(ne, wd1a_ref[...], preferred_element_type=jnp.float32)
        + jnp.dot(lo, wd1b_ref[...], preferred_element_type=jnp.float32), 0.0)
    out_ref[...] = jnp.dot(h, wd2_ref[...], preferred_element_type=jnp.float32)


def _decode(ne, g0, g1, wu1, wu2a, wu2b, wd1a, wd1b, wd2p):
    row = pl.BlockSpec((RB, L), lambda i: (i, 0))
    half = pl.BlockSpec((RB, H), lambda i: (i, 0))
    wL = pl.BlockSpec((L, L), lambda i: (0, 0))
    wH = pl.BlockSpec((H, L), lambda i: (0, 0))
    return pl.pallas_call(
        _dec_body,
        grid=(NSTEPS,),
        in_specs=[row, half, half, wL, wH, wH, wL, wL, wL],
        out_specs=row,
        out_shape=jax.ShapeDtypeStruct((N, L), jnp.float32),
    )(ne, g0, g1, wu1, wu2a, wu2b, wd1a, wd1b, wd2p)


def kernel(node_features, edge_features, latent_features, edge_index,
           W_node, W_edge, W_msg, W_upd, W_dec1, W_dec2):
    nf_c = node_features.astype(jnp.float32)[:, None]
    lat = latent_features.astype(jnp.float32)
    ne, a3, b3, vpn = _encode(
        nf_c, lat, W_node[0:1], W_node[1:], W_edge,
        W_msg[0:L], W_msg[L:2 * L], W_msg[2 * L:])
    a2 = a3.reshape(2 * N, H)
    b2 = b3.reshape(2 * N, H)
    vpnr = jnp.stack([vpn[0, :H], vpn[1, :H], vpn[0, H:], vpn[1, H:]])
    src = edge_index[0].astype(jnp.int32).reshape(NS, NCH, C)
    dst = edge_index[1].astype(jnp.int32).reshape(NS, NCH, C)
    edata = jnp.stack([src, dst], axis=2).reshape(-1)
    ef = edge_features.astype(jnp.float32)
    zer = jnp.zeros((RPT, H), jnp.float32)
    g0, g1 = _edge_call(a2, b2, edata, ef, vpnr, zer)
    wd2p = jnp.pad(W_dec2, ((0, 0), (0, L - 1)))
    outp = _decode(ne, g0, g1, W_upd[:L], W_upd[L:L + H], W_upd[L + H:],
                   W_dec1[:L], W_dec1[L:], wd2p)
    return outp[:, :1]
